# Initial kernel scaffold; baseline (speedup 1.0000x reference)
import jax
import jax.numpy as jnp
from jax.experimental import pallas as pl


def kernel(x, edge_index):
    raise NotImplementedError("write your pallas kernel here")


# speedup vs baseline: 7.7523x; 7.7523x over previous
import functools

import jax
import jax.numpy as jnp
from jax import lax
from jax.experimental import pallas as pl
from jax.experimental.pallas import tpu as pltpu
from jax.experimental.pallas import tpu_sc as plsc

N_NODES = 10000
D_FEAT = 128
N_EDGES = 320000

NUM_CORES = 2
NUM_SUBCORES = 16
NUM_WORKERS = NUM_CORES * NUM_SUBCORES

CHUNK = 80
EDGES_PER_TILE = N_EDGES // NUM_WORKERS
CHUNKS_PER_TILE = EDGES_PER_TILE // CHUNK
ROW_CHUNK = 80
N_ROW_CHUNKS = N_NODES // ROW_CHUNK


def _sc_kernel_body(src_hbm, dst_hbm, x_hbm, part_hbm,
                    acc, srcv, dstv, rows, gsem):
    c = lax.axis_index("c")
    s = lax.axis_index("s")
    wid = c * NUM_SUBCORES + s

    def zero_row(r, _):
        for k in range(D_FEAT // 16):
            rows[r, pl.ds(k * 16, 16)] = jnp.zeros((16,), jnp.float32)
        return _
    lax.fori_loop(0, ROW_CHUNK, zero_row, None)

    for k in range((N_ROW_CHUNKS + NUM_SUBCORES - 1) // NUM_SUBCORES):
        j = s + k * NUM_SUBCORES

        @pl.when(j < N_ROW_CHUNKS)
        def _():
            pltpu.sync_copy(rows, acc.at[pl.ds(j * ROW_CHUNK, ROW_CHUNK)])

    pltpu.sync_copy(src_hbm.at[wid], srcv)
    pltpu.sync_copy(dst_hbm.at[wid], dstv)

    plsc.subcore_barrier()

    def body(j, _):
        pltpu.async_copy(x_hbm.at[srcv.at[j]], rows, gsem).wait()
        pltpu.sync_copy(rows, acc.at[dstv.at[j]], add=True)
        return _
    lax.fori_loop(0, CHUNKS_PER_TILE, body, None)

    plsc.subcore_barrier()

    for k in range((N_ROW_CHUNKS + NUM_SUBCORES - 1) // NUM_SUBCORES):
        j = s + k * NUM_SUBCORES

        @pl.when(j < N_ROW_CHUNKS)
        def _():
            pltpu.sync_copy(acc.at[pl.ds(j * ROW_CHUNK, ROW_CHUNK)], rows)
            pltpu.sync_copy(rows, part_hbm.at[c, pl.ds(j * ROW_CHUNK, ROW_CHUNK)])


_sc_scatter_gather = functools.partial(
    pl.kernel,
    out_type=jax.ShapeDtypeStruct((NUM_CORES, N_NODES, D_FEAT), jnp.float32),
    mesh=plsc.VectorSubcoreMesh(core_axis_name="c", subcore_axis_name="s"),
    scratch_types=[
        pltpu.VMEM_SHARED((N_NODES, D_FEAT), jnp.float32),
        pltpu.VMEM((CHUNKS_PER_TILE, CHUNK), jnp.int32),
        pltpu.VMEM((CHUNKS_PER_TILE, CHUNK), jnp.int32),
        pltpu.VMEM((CHUNK, D_FEAT), jnp.float32),
        pltpu.SemaphoreType.DMA,
    ],
)(_sc_kernel_body)


def _add_body(a_ref, b_ref, o_ref):
    o_ref[...] = a_ref[0] + b_ref[0]


def _combine(partials):
    rows_per_blk = N_NODES // 10
    return pl.pallas_call(
        _add_body,
        out_shape=jax.ShapeDtypeStruct((N_NODES, D_FEAT), jnp.float32),
        grid=(10,),
        in_specs=[
            pl.BlockSpec((1, rows_per_blk, D_FEAT), lambda i: (0, i, 0)),
            pl.BlockSpec((1, rows_per_blk, D_FEAT), lambda i: (1, i, 0)),
        ],
        out_specs=pl.BlockSpec((rows_per_blk, D_FEAT), lambda i: (i, 0)),
    )(partials, partials)


def kernel(x, edge_index):
    src = edge_index[0].astype(jnp.int32).reshape(
        NUM_WORKERS, CHUNKS_PER_TILE, CHUNK)
    dst = edge_index[1].astype(jnp.int32).reshape(
        NUM_WORKERS, CHUNKS_PER_TILE, CHUNK)
    partials = _sc_scatter_gather(src, dst, x)
    return _combine(partials)
